# baseline (speedup 1.0000x reference)
import functools

import jax
import jax.numpy as jnp
from jax import lax
from jax.experimental import pallas as pl
from jax.experimental.pallas import tpu as pltpu
from jax.experimental.pallas import tpu_sc as plsc

_DIM = 4096
_BATCH = 8192
_CTRL_MASK = 2048 | 64

_SC_ROWS = 128
_TC_ROWS = _BATCH - _SC_ROWS
_TC_BLOCK = 128

_NW = 32
_RPW = _SC_ROWS // _NW
_CR = 4
_NCHUNK = _RPW // _CR
_CHUNK = _CR * _DIM
_NB = 1
_NGROUP = _NCHUNK // _NB

_SEG_STARTS = [2048 + 64 + 128 * k for k in range(16)]

_mesh = plsc.VectorSubcoreMesh(core_axis_name="c", subcore_axis_name="s")

_SCRATCH = [pltpu.VMEM((_CHUNK,), jnp.float32) for _ in range(_NB)]
_SCRATCH += [pltpu.SemaphoreType.DMA for _ in range(2 * _NB)]


@functools.partial(
    pl.kernel,
    mesh=_mesh,
    out_type=jax.ShapeDtypeStruct((_SC_ROWS * _DIM,), jnp.float32),
    scratch_types=_SCRATCH,
)
def _sc_perm(state_hbm, out_hbm, *scratch):
    bufs = scratch[:_NB]
    isems = scratch[_NB:2 * _NB]
    osems = scratch[2 * _NB:]
    wid = lax.axis_index("s") * 2 + lax.axis_index("c")
    base = wid * _RPW * _DIM
    swap_idx = (lax.iota(jnp.int32, 16) ^ 1).reshape(16, 1)
    dnums = lax.GatherDimensionNumbers(
        offset_dims=(), collapsed_slice_dims=(0,), start_index_map=(0,)
    )

    def pair_swap(vec):
        return lax.gather(
            vec,
            swap_idx,
            dnums,
            slice_sizes=(1,),
            mode=lax.GatherScatterMode.PROMISE_IN_BOUNDS,
        )

    def in_copy(ci, b):
        return pltpu.make_async_copy(
            state_hbm.at[pl.ds(base + ci * _CHUNK, _CHUNK)],
            bufs[b],
            isems[b],
        )

    def out_copy(ci, b):
        return pltpu.make_async_copy(
            bufs[b],
            out_hbm.at[pl.ds(base + ci * _CHUNK, _CHUNK)],
            osems[b],
        )

    def compute(b):
        for r in range(_CR):
            rbase = r * _DIM
            for seg in _SEG_STARTS:
                for v in range(4):
                    c = rbase + seg + 16 * v
                    bufs[b][pl.ds(c, 16)] = pair_swap(bufs[b][pl.ds(c, 16)])

    def refill(cr, br):
        out_copy(cr - _NB, br).wait()
        in_copy(cr, br).start()

    for b in range(_NB):
        in_copy(b, b).start()

    for b in range(_NB):
        in_copy(b, b).wait()
        compute(b)
        out_copy(b, b).start()
        if _NGROUP > 1 and b + 2 >= _NB:
            refill(b + 2, (b + 2) % _NB)

    def group_body(g, _):
        for b in range(_NB):
            ci = g * _NB + b
            in_copy(ci, b).wait()
            compute(b)
            out_copy(ci, b).start()
            refill(ci + 2, (b + 2) % _NB)
        return 0

    lax.fori_loop(1, _NGROUP - 1, group_body, 0)

    for b in (range(_NB) if _NGROUP > 1 else ()):
        ci = _NCHUNK - _NB + b
        in_copy(ci, b).wait()
        compute(b)
        out_copy(ci, b).start()
        if b + 2 < _NB:
            refill(ci + 2, (b + 2) % _NB)

    for b in range(_NB):
        out_copy(_NCHUNK - _NB + b, b).wait()


def _tc_body(x_ref, o_ref):
    x = x_ref[...]
    cols = lax.broadcasted_iota(jnp.int32, x.shape, 1)
    swapped = (cols & _CTRL_MASK) == _CTRL_MASK
    odd = (cols & 1) == 1
    left = jnp.roll(x, -1, axis=1)
    right = jnp.roll(x, 1, axis=1)
    o_ref[...] = jnp.where(swapped, jnp.where(odd, right, left), x)


_tc_perm = pl.pallas_call(
    _tc_body,
    grid=(_TC_ROWS // _TC_BLOCK,),
    in_specs=[
        pl.BlockSpec(
            (_TC_BLOCK, _DIM),
            lambda i: (i + _SC_ROWS // _TC_BLOCK, 0),
        )
    ],
    out_specs=pl.BlockSpec((_TC_BLOCK, _DIM), lambda i: (i, 0)),
    out_shape=jax.ShapeDtypeStruct((_TC_ROWS, _DIM), jnp.float32),
)


def kernel(state, M):
    del M
    flat = state.reshape(_BATCH * _DIM)
    sc_out = _sc_perm(flat).reshape(_SC_ROWS, _DIM)
    tc_out = _tc_perm(state)
    return jnp.concatenate([sc_out, tc_out], axis=0)


# speedup vs baseline: 3.3682x; 3.3682x over previous
import jax
import jax.numpy as jnp
from jax import lax
from jax.experimental import pallas as pl

_DIM = 4096
_BATCH = 8192
_BLOCK = 256
_CTRL_MASK = 2048 | 64


def _body(x_ref, o_ref):
    x = x_ref[...]
    cols = lax.broadcasted_iota(jnp.int32, x.shape, 1)
    swapped = (cols & _CTRL_MASK) == _CTRL_MASK
    odd = (cols & 1) == 1
    left = jnp.roll(x, -1, axis=1)
    right = jnp.roll(x, 1, axis=1)
    o_ref[...] = jnp.where(swapped, jnp.where(odd, right, left), x)


_perm = pl.pallas_call(
    _body,
    grid=(_BATCH // _BLOCK,),
    in_specs=[pl.BlockSpec((_BLOCK, _DIM), lambda i: (i, 0))],
    out_specs=pl.BlockSpec((_BLOCK, _DIM), lambda i: (i, 0)),
    out_shape=jax.ShapeDtypeStruct((_BATCH, _DIM), jnp.float32),
)


def kernel(state, M):
    del M
    return _perm(state)


# speedup vs baseline: 3.4834x; 1.0342x over previous
import jax
import jax.numpy as jnp
from jax import lax
from jax.experimental import pallas as pl

_DIM = 4096
_BATCH = 8192
_BLOCK = 512
_CTRL_MASK = 2048 | 64


def _body(x_ref, o_ref):
    x = x_ref[...]
    cols = lax.broadcasted_iota(jnp.int32, x.shape, 1)
    swapped = (cols & _CTRL_MASK) == _CTRL_MASK
    odd = (cols & 1) == 1
    left = jnp.roll(x, -1, axis=1)
    right = jnp.roll(x, 1, axis=1)
    o_ref[...] = jnp.where(swapped, jnp.where(odd, right, left), x)


_perm = pl.pallas_call(
    _body,
    grid=(_BATCH // _BLOCK,),
    in_specs=[pl.BlockSpec((_BLOCK, _DIM), lambda i: (i, 0))],
    out_specs=pl.BlockSpec((_BLOCK, _DIM), lambda i: (i, 0)),
    out_shape=jax.ShapeDtypeStruct((_BATCH, _DIM), jnp.float32),
)


def kernel(state, M):
    del M
    return _perm(state)
